# initial kernel scaffold (unmeasured)
import jax
import jax.numpy as jnp
from jax import lax
from jax.experimental import pallas as pl
from jax.experimental.pallas import tpu as pltpu

N_DEV = 4


def kernel(A, B):
    A = A.astype(jnp.bfloat16)
    B = B.astype(jnp.bfloat16)
    m, _ = A.shape
    _, n = B.shape
    m_chunk = m // N_DEV

    def body(a_ref, b_ref, out_ref, comm_ref, send_sems, recv_sems):
        my = lax.axis_index("i")
        left = (my + N_DEV - 1) % N_DEV
        right = (my + 1) % N_DEV

        def partial(c):
            return jnp.dot(
                a_ref[pl.ds(c * m_chunk, m_chunk), :],
                b_ref[...],
                preferred_element_type=jnp.float32,
            )

        barrier_sem = pltpu.get_barrier_semaphore()

        def barrier():
            for nbr in (left, right):
                pl.semaphore_signal(
                    barrier_sem,
                    inc=1,
                    device_id=(nbr,),
                    device_id_type=pl.DeviceIdType.MESH,
                )
            pl.semaphore_wait(barrier_sem, 2)

        comm_ref[0, :, :] = partial((my + N_DEV - 1) % N_DEV)
        barrier()

        for h in range(N_DEV - 1):
            send_slot, recv_slot = h % 2, (h + 1) % 2
            rdma = pltpu.make_async_remote_copy(
                src_ref=comm_ref.at[send_slot],
                dst_ref=comm_ref.at[recv_slot],
                send_sem=send_sems.at[h],
                recv_sem=recv_sems.at[h],
                device_id=(right,),
                device_id_type=pl.DeviceIdType.MESH,
            )
            rdma.start()
            p = partial((my + N_DEV - 2 - h) % N_DEV)
            rdma.wait()
            if h < N_DEV - 2:
                comm_ref[recv_slot, :, :] = comm_ref[recv_slot, :, :] + p
            else:
                out_ref[...] = comm_ref[recv_slot, :, :] + p
            barrier()

    return pl.pallas_call(
        body,
        out_shape=jax.ShapeDtypeStruct((m_chunk, n), jnp.float32),
        in_specs=[
            pl.BlockSpec(memory_space=pltpu.VMEM),
            pl.BlockSpec(memory_space=pltpu.VMEM),
        ],
        out_specs=pl.BlockSpec(memory_space=pltpu.VMEM),
        scratch_shapes=[
            pltpu.VMEM((2, m_chunk, n), jnp.float32),
            pltpu.SemaphoreType.DMA((N_DEV - 1,)),
            pltpu.SemaphoreType.DMA((N_DEV - 1,)),
        ],
        compiler_params=pltpu.CompilerParams(collective_id=0),
    )(A, B)


# baseline (device time: 350648 ns/iter reference)
import jax
import jax.numpy as jnp
from jax import lax
from jax.experimental import pallas as pl
from jax.experimental.pallas import tpu as pltpu

N_DEV = 4
N_TILE = 512


def kernel(A, B):
    A = A.astype(jnp.bfloat16)
    B = B.astype(jnp.bfloat16)
    m, k = A.shape
    _, n = B.shape
    m_chunk = m // N_DEV
    n_strips = n // N_TILE

    def body(a_hbm, b_ref, out_ref, a_buf, comm_ref, copy_sem, send_sems,
             recv_sems):
        my = lax.axis_index("i")
        left = (my + N_DEV - 1) % N_DEV
        right = (my + 1) % N_DEV

        def chunk_idx(s):
            return (my + N_DEV - 1 - s) % N_DEV

        def fetch(s):
            return pltpu.make_async_copy(
                a_hbm.at[pl.ds(chunk_idx(s) * m_chunk, m_chunk), :],
                a_buf,
                copy_sem,
            )

        def strip(j):
            return pl.ds(j * N_TILE, N_TILE)

        def partial_strip(j):
            return jnp.dot(
                a_buf[...],
                b_ref[:, strip(j)],
                preferred_element_type=jnp.float32,
            )

        barrier_sem = pltpu.get_barrier_semaphore()

        def barrier():
            for nbr in (left, right):
                pl.semaphore_signal(
                    barrier_sem,
                    inc=1,
                    device_id=(nbr,),
                    device_id_type=pl.DeviceIdType.MESH,
                )
            pl.semaphore_wait(barrier_sem, 2)

        fetch(0).start()
        fetch(0).wait()
        for j in range(n_strips):
            comm_ref[0, :, strip(j)] = partial_strip(j).astype(jnp.bfloat16)
        fetch(1).start()
        barrier()

        for h in range(N_DEV - 1):
            send_slot, recv_slot = h % 2, (h + 1) % 2
            rdmas = [
                pltpu.make_async_remote_copy(
                    src_ref=comm_ref.at[send_slot, :, strip(j)],
                    dst_ref=comm_ref.at[recv_slot, :, strip(j)],
                    send_sem=send_sems.at[h, j],
                    recv_sem=recv_sems.at[h, j],
                    device_id=(right,),
                    device_id_type=pl.DeviceIdType.MESH,
                )
                for j in range(n_strips)
            ]
            for r in rdmas:
                r.start()
            fetch(h + 1).wait()
            for j in range(n_strips):
                t = partial_strip(j)
                rdmas[j].wait_recv()
                acc = (
                    comm_ref[recv_slot, :, strip(j)].astype(jnp.float32) + t
                ).astype(jnp.bfloat16)
                if h < N_DEV - 2:
                    comm_ref[recv_slot, :, strip(j)] = acc
                else:
                    out_ref[:, strip(j)] = acc
            if h + 2 < N_DEV:
                fetch(h + 2).start()
            for r in rdmas:
                r.wait_send()
            barrier()

    return pl.pallas_call(
        body,
        out_shape=jax.ShapeDtypeStruct((m_chunk, n), jnp.bfloat16),
        in_specs=[
            pl.BlockSpec(memory_space=pltpu.MemorySpace.HBM),
            pl.BlockSpec(memory_space=pltpu.VMEM),
        ],
        out_specs=pl.BlockSpec(memory_space=pltpu.VMEM),
        scratch_shapes=[
            pltpu.VMEM((m_chunk, k), jnp.bfloat16),
            pltpu.VMEM((2, m_chunk, n), jnp.bfloat16),
            pltpu.SemaphoreType.DMA,
            pltpu.SemaphoreType.DMA((N_DEV - 1, n // N_TILE)),
            pltpu.SemaphoreType.DMA((N_DEV - 1, n // N_TILE)),
        ],
        compiler_params=pltpu.CompilerParams(
            collective_id=0,
            vmem_limit_bytes=60 * 1024 * 1024,
        ),
    )(A, B)


# device time: 334868 ns/iter; 1.0471x vs baseline; 1.0471x over previous
import jax
import jax.numpy as jnp
from jax import lax
from jax.experimental import pallas as pl
from jax.experimental.pallas import tpu as pltpu

N_DEV = 4
N_TILE = 512


def kernel(A, B):
    A = A.astype(jnp.bfloat16)
    B = B.astype(jnp.bfloat16)
    m, k = A.shape
    _, n = B.shape
    m_chunk = m // N_DEV
    n_strips = n // N_TILE

    def body(a_hbm, b_ref, out_ref, a_buf, comm_ref, copy_sem, send_sems,
             recv_sems):
        my = lax.axis_index("i")
        left = (my + N_DEV - 1) % N_DEV
        right = (my + 1) % N_DEV

        def chunk_idx(s):
            return (my + N_DEV - 1 - s) % N_DEV

        def fetch(s):
            return pltpu.make_async_copy(
                a_hbm.at[pl.ds(chunk_idx(s) * m_chunk, m_chunk), :],
                a_buf,
                copy_sem,
            )

        def strip(j):
            return pl.ds(j * N_TILE, N_TILE)

        def partial_strip(j):
            return jnp.dot(
                a_buf[...],
                b_ref[:, strip(j)],
                preferred_element_type=jnp.float32,
            )

        barrier_sem = pltpu.get_barrier_semaphore()

        def barrier():
            for nbr in (left, right):
                pl.semaphore_signal(
                    barrier_sem,
                    inc=1,
                    device_id=(nbr,),
                    device_id_type=pl.DeviceIdType.MESH,
                )
            pl.semaphore_wait(barrier_sem, 2)

        def hop_rdmas(h, send_slot, recv_slot):
            return [
                pltpu.make_async_remote_copy(
                    src_ref=comm_ref.at[send_slot, :, strip(j)],
                    dst_ref=comm_ref.at[recv_slot, :, strip(j)],
                    send_sem=send_sems.at[h, j],
                    recv_sem=recv_sems.at[h, j],
                    device_id=(right,),
                    device_id_type=pl.DeviceIdType.MESH,
                )
                for j in range(n_strips)
            ]

        fetch(0).start()
        fetch(0).wait()
        barrier()
        rdmas0 = hop_rdmas(0, 0, 1)
        for j in range(n_strips):
            comm_ref[0, :, strip(j)] = partial_strip(j).astype(jnp.bfloat16)
            rdmas0[j].start()
        fetch(1).start()

        for h in range(N_DEV - 1):
            send_slot, recv_slot = h % 2, (h + 1) % 2
            rdmas = rdmas0 if h == 0 else hop_rdmas(h, send_slot, recv_slot)
            if h > 0:
                for r in rdmas:
                    r.start()
            fetch(h + 1).wait()
            for j in range(n_strips):
                t = partial_strip(j)
                rdmas[j].wait_recv()
                acc = (
                    comm_ref[recv_slot, :, strip(j)].astype(jnp.float32) + t
                ).astype(jnp.bfloat16)
                if h < N_DEV - 2:
                    comm_ref[recv_slot, :, strip(j)] = acc
                else:
                    out_ref[:, strip(j)] = acc
            if h + 2 < N_DEV:
                fetch(h + 2).start()
            for r in rdmas:
                r.wait_send()
            barrier()

    return pl.pallas_call(
        body,
        out_shape=jax.ShapeDtypeStruct((m_chunk, n), jnp.bfloat16),
        in_specs=[
            pl.BlockSpec(memory_space=pltpu.MemorySpace.HBM),
            pl.BlockSpec(memory_space=pltpu.VMEM),
        ],
        out_specs=pl.BlockSpec(memory_space=pltpu.VMEM),
        scratch_shapes=[
            pltpu.VMEM((m_chunk, k), jnp.bfloat16),
            pltpu.VMEM((2, m_chunk, n), jnp.bfloat16),
            pltpu.SemaphoreType.DMA,
            pltpu.SemaphoreType.DMA((N_DEV - 1, n // N_TILE)),
            pltpu.SemaphoreType.DMA((N_DEV - 1, n // N_TILE)),
        ],
        compiler_params=pltpu.CompilerParams(
            collective_id=0,
            vmem_limit_bytes=60 * 1024 * 1024,
        ),
    )(A, B)


# device time: 329925 ns/iter; 1.0628x vs baseline; 1.0150x over previous
import jax
import jax.numpy as jnp
from jax import lax
from jax.experimental import pallas as pl
from jax.experimental.pallas import tpu as pltpu

N_DEV = 4
N_TILE = 512
N_STRIPS = 4096 // N_TILE


def kernel(A, B):
    A = A.astype(jnp.bfloat16)
    B = B.astype(jnp.bfloat16)
    m, k = A.shape
    _, n = B.shape
    m_chunk = m // N_DEV
    n_strips = n // N_TILE

    def body(a_hbm, b_ref, out_ref, a_buf, comm_ref, copy_sem, send_sems,
             recv_sems, credit_sems):
        my = lax.axis_index("i")
        left = (my + N_DEV - 1) % N_DEV
        right = (my + 1) % N_DEV

        def chunk_idx(s):
            return (my + N_DEV - 1 - s) % N_DEV

        def fetch(s):
            return pltpu.make_async_copy(
                a_hbm.at[pl.ds(chunk_idx(s) * m_chunk, m_chunk), :],
                a_buf,
                copy_sem,
            )

        def strip(j):
            return pl.ds(j * N_TILE, N_TILE)

        def partial_strip(j):
            return jnp.dot(
                a_buf[...],
                b_ref[:, strip(j)],
                preferred_element_type=jnp.float32,
            )

        rdmas = [
            [
                pltpu.make_async_remote_copy(
                    src_ref=comm_ref.at[h % 2, :, strip(j)],
                    dst_ref=comm_ref.at[(h + 1) % 2, :, strip(j)],
                    send_sem=send_sems.at[h, j],
                    recv_sem=recv_sems.at[h, j],
                    device_id=(right,),
                    device_id_type=pl.DeviceIdType.MESH,
                )
                for j in range(n_strips)
            ]
            for h in range(N_DEV - 1)
        ]

        barrier_sem = pltpu.get_barrier_semaphore()

        def barrier():
            for nbr in (left, right):
                pl.semaphore_signal(
                    barrier_sem,
                    inc=1,
                    device_id=(nbr,),
                    device_id_type=pl.DeviceIdType.MESH,
                )
            pl.semaphore_wait(barrier_sem, 2)

        fetch(0).start()
        fetch(0).wait()
        barrier()
        for j in range(n_strips):
            comm_ref[0, :, strip(j)] = partial_strip(j).astype(jnp.bfloat16)
            rdmas[0][j].start()
        fetch(1).start()

        for h in range(N_DEV - 1):
            recv_slot = (h + 1) % 2
            fetch(h + 1).wait()
            for j in range(n_strips):
                t = partial_strip(j)
                rdmas[h][j].wait_recv()
                acc = (
                    comm_ref[recv_slot, :, strip(j)].astype(jnp.float32) + t
                ).astype(jnp.bfloat16)
                if h < N_DEV - 2:
                    comm_ref[recv_slot, :, strip(j)] = acc
                    rdmas[h][j].wait_send()
                    pl.semaphore_signal(
                        credit_sems.at[h, j],
                        inc=1,
                        device_id=(left,),
                        device_id_type=pl.DeviceIdType.MESH,
                    )
                    pl.semaphore_wait(credit_sems.at[h, j], 1)
                    rdmas[h + 1][j].start()
                else:
                    out_ref[:, strip(j)] = acc
                    rdmas[h][j].wait_send()
            if h + 2 < N_DEV:
                fetch(h + 2).start()
        barrier()

    return pl.pallas_call(
        body,
        out_shape=jax.ShapeDtypeStruct((m_chunk, n), jnp.bfloat16),
        in_specs=[
            pl.BlockSpec(memory_space=pltpu.MemorySpace.HBM),
            pl.BlockSpec(memory_space=pltpu.VMEM),
        ],
        out_specs=pl.BlockSpec(memory_space=pltpu.VMEM),
        scratch_shapes=[
            pltpu.VMEM((m_chunk, k), jnp.bfloat16),
            pltpu.VMEM((2, m_chunk, n), jnp.bfloat16),
            pltpu.SemaphoreType.DMA,
            pltpu.SemaphoreType.DMA((N_DEV - 1, N_STRIPS)),
            pltpu.SemaphoreType.DMA((N_DEV - 1, N_STRIPS)),
            pltpu.SemaphoreType.REGULAR((N_DEV - 2, N_STRIPS)),
        ],
        compiler_params=pltpu.CompilerParams(
            collective_id=0,
            vmem_limit_bytes=60 * 1024 * 1024,
        ),
    )(A, B)


# device time: 329810 ns/iter; 1.0632x vs baseline; 1.0003x over previous
import jax
import jax.numpy as jnp
from jax import lax
from jax.experimental import pallas as pl
from jax.experimental.pallas import tpu as pltpu

N_DEV = 4
N_TILE = 512
N_STRIPS = 4096 // N_TILE


def kernel(A, B):
    A = A.astype(jnp.bfloat16)
    B = B.astype(jnp.bfloat16)
    m, k = A.shape
    _, n = B.shape
    m_chunk = m // N_DEV
    n_strips = n // N_TILE

    def body(a_hbm, b_ref, out_ref, a_buf, comm_ref, copy_sem, send_sems,
             recv_sems, credit_sems):
        my = lax.axis_index("i")
        left = (my + N_DEV - 1) % N_DEV
        right = (my + 1) % N_DEV

        def chunk_idx(s):
            return (my + N_DEV - 1 - s) % N_DEV

        def fetch(s):
            return pltpu.make_async_copy(
                a_hbm.at[pl.ds(chunk_idx(s) * m_chunk, m_chunk), :],
                a_buf,
                copy_sem,
            )

        def strip(j):
            return pl.ds(j * N_TILE, N_TILE)

        def partial_strip(j):
            return jnp.dot(
                a_buf[...],
                b_ref[:, strip(j)],
                preferred_element_type=jnp.float32,
            )

        rdmas = [
            [
                pltpu.make_async_remote_copy(
                    src_ref=comm_ref.at[h % 2, j],
                    dst_ref=comm_ref.at[(h + 1) % 2, j],
                    send_sem=send_sems.at[h, j],
                    recv_sem=recv_sems.at[h, j],
                    device_id=(right,),
                    device_id_type=pl.DeviceIdType.MESH,
                )
                for j in range(n_strips)
            ]
            for h in range(N_DEV - 1)
        ]

        barrier_sem = pltpu.get_barrier_semaphore()

        def barrier():
            for nbr in (left, right):
                pl.semaphore_signal(
                    barrier_sem,
                    inc=1,
                    device_id=(nbr,),
                    device_id_type=pl.DeviceIdType.MESH,
                )
            pl.semaphore_wait(barrier_sem, 2)

        fetch(0).start()
        fetch(0).wait()
        barrier()
        for j in range(n_strips):
            comm_ref[0, j] = partial_strip(j).astype(jnp.bfloat16)
            rdmas[0][j].start()
        fetch(1).start()

        for h in range(N_DEV - 1):
            recv_slot = (h + 1) % 2
            fetch(h + 1).wait()
            for j in range(n_strips):
                t = partial_strip(j)
                rdmas[h][j].wait_recv()
                acc = (
                    comm_ref[recv_slot, j].astype(jnp.float32) + t
                ).astype(jnp.bfloat16)
                if h < N_DEV - 2:
                    comm_ref[recv_slot, j] = acc
                    rdmas[h][j].wait_send()
                    pl.semaphore_signal(
                        credit_sems.at[h, j],
                        inc=1,
                        device_id=(left,),
                        device_id_type=pl.DeviceIdType.MESH,
                    )
                    pl.semaphore_wait(credit_sems.at[h, j], 1)
                    rdmas[h + 1][j].start()
                else:
                    out_ref[:, strip(j)] = acc
                    rdmas[h][j].wait_send()
            if h + 2 < N_DEV:
                fetch(h + 2).start()
        barrier()

    return pl.pallas_call(
        body,
        out_shape=jax.ShapeDtypeStruct((m_chunk, n), jnp.bfloat16),
        in_specs=[
            pl.BlockSpec(memory_space=pltpu.MemorySpace.HBM),
            pl.BlockSpec(memory_space=pltpu.VMEM),
        ],
        out_specs=pl.BlockSpec(memory_space=pltpu.VMEM),
        scratch_shapes=[
            pltpu.VMEM((m_chunk, k), jnp.bfloat16),
            pltpu.VMEM((2, n // N_TILE, m_chunk, N_TILE), jnp.bfloat16),
            pltpu.SemaphoreType.DMA,
            pltpu.SemaphoreType.DMA((N_DEV - 1, N_STRIPS)),
            pltpu.SemaphoreType.DMA((N_DEV - 1, N_STRIPS)),
            pltpu.SemaphoreType.REGULAR((N_DEV - 2, N_STRIPS)),
        ],
        compiler_params=pltpu.CompilerParams(
            collective_id=0,
            vmem_limit_bytes=60 * 1024 * 1024,
        ),
    )(A, B)


# device time: 292953 ns/iter; 1.1969x vs baseline; 1.1258x over previous
import jax
import jax.numpy as jnp
from jax import lax
from jax.experimental import pallas as pl
from jax.experimental.pallas import tpu as pltpu

N_DEV = 4
N_TILE = 512
N_STRIPS = 4096 // N_TILE


def kernel(A, B):
    m, k = A.shape
    _, n = B.shape
    m_chunk = m // N_DEV
    m_half = m_chunk // 2
    n_strips = n // N_TILE

    def body(a_hbm, b_hbm, out_hbm, a_stage, a_buf, b_stage, b_buf,
             comm_ref, a_sem, b_sem, out_sems, send_sems, recv_sems,
             credit_sems):
        my = lax.axis_index("i")
        left = (my + N_DEV - 1) % N_DEV
        right = (my + 1) % N_DEV

        def chunk_idx(s):
            return (my + N_DEV - 1 - s) % N_DEV

        def fetch_a(s, half):
            return pltpu.make_async_copy(
                a_hbm.at[
                    pl.ds(chunk_idx(s) * m_chunk + half * m_half, m_half), :
                ],
                a_stage,
                a_sem,
            )

        def convert_a(s, half):
            a_buf[s % 2, pl.ds(half * m_half, m_half), :] = a_stage[
                ...
            ].astype(jnp.bfloat16)

        def strip(j):
            return pl.ds(j * N_TILE, N_TILE)

        def fetch_b(j):
            return pltpu.make_async_copy(
                b_hbm.at[:, strip(j)], b_stage, b_sem
            )

        def convert_b(j):
            b_buf[pl.ds(0, k // 2), strip(j)] = b_stage[
                pl.ds(0, k // 2), :
            ].astype(jnp.bfloat16)
            b_buf[pl.ds(k // 2, k // 2), strip(j)] = b_stage[
                pl.ds(k // 2, k // 2), :
            ].astype(jnp.bfloat16)

        def partial_strip(s, j):
            return jnp.dot(
                a_buf[s % 2],
                b_buf[:, strip(j)],
                preferred_element_type=jnp.float32,
            )

        rdmas = [
            [
                pltpu.make_async_remote_copy(
                    src_ref=comm_ref.at[h % 2, j],
                    dst_ref=comm_ref.at[(h + 1) % 2, j],
                    send_sem=send_sems.at[h, j],
                    recv_sem=recv_sems.at[h, j],
                    device_id=(right,),
                    device_id_type=pl.DeviceIdType.MESH,
                )
                for j in range(n_strips)
            ]
            for h in range(N_DEV - 1)
        ]

        def store_out(j):
            return pltpu.make_async_copy(
                comm_ref.at[1, j], out_hbm.at[:, strip(j)], out_sems.at[j]
            )

        barrier_sem = pltpu.get_barrier_semaphore()

        def barrier():
            for nbr in (left, right):
                pl.semaphore_signal(
                    barrier_sem,
                    inc=1,
                    device_id=(nbr,),
                    device_id_type=pl.DeviceIdType.MESH,
                )
            pl.semaphore_wait(barrier_sem, 2)

        fetch_b(0).start()
        fetch_a(0, 0).start()
        fetch_a(0, 0).wait()
        convert_a(0, 0)
        fetch_a(0, 1).start()
        fetch_a(0, 1).wait()
        convert_a(0, 1)
        fetch_a(1, 0).start()
        barrier()

        for j in range(n_strips):
            fetch_b(j).wait()
            convert_b(j)
            if j + 1 < n_strips:
                fetch_b(j + 1).start()
            comm_ref[0, j] = partial_strip(0, j).astype(jnp.bfloat16)
            rdmas[0][j].start()
            if j == 0:
                fetch_a(1, 0).wait()
                convert_a(1, 0)
                fetch_a(1, 1).start()
            elif j == 1:
                fetch_a(1, 1).wait()
                convert_a(1, 1)

        for h in range(N_DEV - 1):
            recv_slot = (h + 1) % 2
            if h + 2 < N_DEV:
                fetch_a(h + 2, 0).start()
            for j in range(n_strips):
                t = partial_strip(h + 1, j)
                rdmas[h][j].wait_recv()
                acc = (
                    comm_ref[recv_slot, j].astype(jnp.float32) + t
                ).astype(jnp.bfloat16)
                if h < N_DEV - 2:
                    comm_ref[recv_slot, j] = acc
                    rdmas[h][j].wait_send()
                    pl.semaphore_signal(
                        credit_sems.at[h, j],
                        inc=1,
                        device_id=(left,),
                        device_id_type=pl.DeviceIdType.MESH,
                    )
                    pl.semaphore_wait(credit_sems.at[h, j], 1)
                    rdmas[h + 1][j].start()
                else:
                    comm_ref[recv_slot, j] = acc
                    store_out(j).start()
                    rdmas[h][j].wait_send()
                if h + 2 < N_DEV:
                    if j == 0:
                        fetch_a(h + 2, 0).wait()
                        convert_a(h + 2, 0)
                        fetch_a(h + 2, 1).start()
                    elif j == 1:
                        fetch_a(h + 2, 1).wait()
                        convert_a(h + 2, 1)
        for j in range(n_strips):
            store_out(j).wait()
        barrier()

    return pl.pallas_call(
        body,
        out_shape=jax.ShapeDtypeStruct((m_chunk, n), jnp.bfloat16),
        in_specs=[
            pl.BlockSpec(memory_space=pltpu.MemorySpace.HBM),
            pl.BlockSpec(memory_space=pltpu.MemorySpace.HBM),
        ],
        out_specs=pl.BlockSpec(memory_space=pltpu.MemorySpace.HBM),
        scratch_shapes=[
            pltpu.VMEM((m_half, k), jnp.float32),
            pltpu.VMEM((2, m_chunk, k), jnp.bfloat16),
            pltpu.VMEM((k, N_TILE), jnp.float32),
            pltpu.VMEM((k, 4096), jnp.bfloat16),
            pltpu.VMEM((2, N_STRIPS, m_chunk, N_TILE), jnp.bfloat16),
            pltpu.SemaphoreType.DMA,
            pltpu.SemaphoreType.DMA,
            pltpu.SemaphoreType.DMA((N_STRIPS,)),
            pltpu.SemaphoreType.DMA((N_DEV - 1, N_STRIPS)),
            pltpu.SemaphoreType.DMA((N_DEV - 1, N_STRIPS)),
            pltpu.SemaphoreType.REGULAR((N_DEV - 2, N_STRIPS)),
        ],
        compiler_params=pltpu.CompilerParams(
            collective_id=0,
            vmem_limit_bytes=62 * 1024 * 1024,
        ),
    )(A, B)


# device time: 164348 ns/iter; 2.1336x vs baseline; 1.7825x over previous
import jax
import jax.numpy as jnp
from jax import lax
from jax.experimental import pallas as pl
from jax.experimental.pallas import tpu as pltpu

N_DEV = 4
N_TILE = 512
N_STRIPS = 4096 // N_TILE


def kernel(A, B):
    m, k = A.shape
    _, n = B.shape
    m_chunk = m // N_DEV
    m_half = m_chunk // 2
    n_strips = n // N_TILE

    def body(a_hbm, b_hbm, out_hbm, a_stage, a_buf, b_stage, b_buf,
             comm_ref, a_sem, b_sem, out_sems, send_sems, recv_sems,
             credit_sems):
        my = lax.axis_index("i")
        left = (my + N_DEV - 1) % N_DEV
        right = (my + 1) % N_DEV

        c_a = (my + N_DEV - 1) % N_DEV
        c_b = (my + 1) % N_DEV
        c_mid = (my + 2) % N_DEV
        c_my = my

        def dot_slot(stage, par):
            return [(0, 1), (2, 2), (1, 0), (2, 2)][stage][par]

        def fetch_a(c, half):
            return pltpu.make_async_copy(
                a_hbm.at[pl.ds(c * m_chunk + half * m_half, m_half), :],
                a_stage,
                a_sem,
            )

        def convert_a(slot, half):
            a_buf[slot, pl.ds(half * m_half, m_half), :] = a_stage[
                ...
            ].astype(jnp.bfloat16)

        def strip(j):
            return pl.ds(j * N_TILE, N_TILE)

        def fetch_b(j):
            return pltpu.make_async_copy(
                b_hbm.at[:, strip(j)], b_stage, b_sem
            )

        def convert_b(j):
            b_buf[pl.ds(0, k // 2), strip(j)] = b_stage[
                pl.ds(0, k // 2), :
            ].astype(jnp.bfloat16)
            b_buf[pl.ds(k // 2, k // 2), strip(j)] = b_stage[
                pl.ds(k // 2, k // 2), :
            ].astype(jnp.bfloat16)

        def partial_strip(stage, j):
            return jnp.dot(
                a_buf[dot_slot(stage, j % 2)],
                b_buf[:, strip(j)],
                preferred_element_type=jnp.float32,
            )

        rdmas = [
            [
                pltpu.make_async_remote_copy(
                    src_ref=comm_ref.at[h % 2, j],
                    dst_ref=comm_ref.at[(h + 1) % 2, j],
                    send_sem=send_sems.at[h, j],
                    recv_sem=recv_sems.at[h, j],
                    device_id=(right if j % 2 == 0 else left,),
                    device_id_type=pl.DeviceIdType.MESH,
                )
                for j in range(n_strips)
            ]
            for h in range(N_DEV - 1)
        ]

        def store_out(j):
            return pltpu.make_async_copy(
                comm_ref.at[1, j], out_hbm.at[:, strip(j)], out_sems.at[j]
            )

        barrier_sem = pltpu.get_barrier_semaphore()

        def barrier():
            for nbr in (left, right):
                pl.semaphore_signal(
                    barrier_sem,
                    inc=1,
                    device_id=(nbr,),
                    device_id_type=pl.DeviceIdType.MESH,
                )
            pl.semaphore_wait(barrier_sem, 2)

        fetch_b(0).start()
        fetch_a(c_a, 0).start()
        fetch_a(c_a, 0).wait()
        convert_a(0, 0)
        fetch_a(c_a, 1).start()
        fetch_a(c_a, 1).wait()
        convert_a(0, 1)
        fetch_a(c_b, 0).start()
        barrier()

        for j in range(n_strips):
            fetch_b(j).wait()
            convert_b(j)
            if j + 1 < n_strips:
                fetch_b(j + 1).start()
            comm_ref[0, j] = partial_strip(0, j).astype(jnp.bfloat16)
            rdmas[0][j].start()
            if j == 0:
                fetch_a(c_b, 0).wait()
                convert_a(1, 0)
                fetch_a(c_b, 1).start()
                fetch_a(c_b, 1).wait()
                convert_a(1, 1)
                fetch_a(c_mid, 0).start()
            elif j == 2:
                fetch_a(c_mid, 0).wait()
                convert_a(2, 0)
                fetch_a(c_mid, 1).start()
            elif j == 3:
                fetch_a(c_mid, 1).wait()
                convert_a(2, 1)

        for h in range(N_DEV - 1):
            recv_slot = (h + 1) % 2
            for j in range(n_strips):
                t = partial_strip(h + 1, j)
                rdmas[h][j].wait_recv()
                acc = (
                    comm_ref[recv_slot, j].astype(jnp.float32) + t
                ).astype(jnp.bfloat16)
                if h < N_DEV - 2:
                    comm_ref[recv_slot, j] = acc
                    rdmas[h][j].wait_send()
                    pl.semaphore_signal(
                        credit_sems.at[h, j],
                        inc=1,
                        device_id=(left if j % 2 == 0 else right,),
                        device_id_type=pl.DeviceIdType.MESH,
                    )
                    pl.semaphore_wait(credit_sems.at[h, j], 1)
                    rdmas[h + 1][j].start()
                else:
                    comm_ref[recv_slot, j] = acc
                    store_out(j).start()
                    rdmas[h][j].wait_send()
                if h == 1:
                    if j == 0:
                        fetch_a(c_my, 0).wait()
                        convert_a(2, 0)
                        fetch_a(c_my, 1).start()
                    elif j == 1:
                        fetch_a(c_my, 1).wait()
                        convert_a(2, 1)
            if h == 0:
                fetch_a(c_my, 0).start()
        for j in range(n_strips):
            store_out(j).wait()
        barrier()

    return pl.pallas_call(
        body,
        out_shape=jax.ShapeDtypeStruct((m_chunk, n), jnp.bfloat16),
        in_specs=[
            pl.BlockSpec(memory_space=pltpu.MemorySpace.HBM),
            pl.BlockSpec(memory_space=pltpu.MemorySpace.HBM),
        ],
        out_specs=pl.BlockSpec(memory_space=pltpu.MemorySpace.HBM),
        scratch_shapes=[
            pltpu.VMEM((m_half, k), jnp.float32),
            pltpu.VMEM((3, m_chunk, k), jnp.bfloat16),
            pltpu.VMEM((k, N_TILE), jnp.float32),
            pltpu.VMEM((k, 4096), jnp.bfloat16),
            pltpu.VMEM((2, N_STRIPS, m_chunk, N_TILE), jnp.bfloat16),
            pltpu.SemaphoreType.DMA,
            pltpu.SemaphoreType.DMA,
            pltpu.SemaphoreType.DMA((N_STRIPS,)),
            pltpu.SemaphoreType.DMA((N_DEV - 1, N_STRIPS)),
            pltpu.SemaphoreType.DMA((N_DEV - 1, N_STRIPS)),
            pltpu.SemaphoreType.REGULAR((N_DEV - 2, N_STRIPS)),
        ],
        compiler_params=pltpu.CompilerParams(
            collective_id=0,
            vmem_limit_bytes=62 * 1024 * 1024,
        ),
    )(A, B)
